# baseline (speedup 1.0000x reference)
import functools

import jax
import jax.numpy as jnp
from jax import lax
from jax.experimental import pallas as pl
from jax.experimental.pallas import tpu as pltpu
from jax.experimental.pallas import tpu_sc as plsc

EMBED = 16
BITS = 32
NUM_WORKERS = 32
CHUNK = 128


def _sc_gather(table, idx, n_tokens):
    b_per_w = n_tokens // NUM_WORKERS
    num_chunks = (b_per_w + CHUNK - 1) // CHUNK
    mesh = plsc.VectorSubcoreMesh(core_axis_name="c", subcore_axis_name="s")

    @functools.partial(
        pl.kernel,
        mesh=mesh,
        out_type=jax.ShapeDtypeStruct((n_tokens, EMBED), jnp.float32),
        scratch_types=[
            pltpu.VMEM((b_per_w,), jnp.int32),
            pltpu.VMEM((b_per_w, EMBED), jnp.float32),
            pltpu.SemaphoreType.DMA,
        ],
        compiler_params=pltpu.CompilerParams(use_tc_tiling_on_sc=False),
    )
    def k(table_hbm, idx_hbm, out_hbm, idx_v, rows_v, sem):
        wid = lax.axis_index("s") * 2 + lax.axis_index("c")
        base = wid * b_per_w
        pltpu.sync_copy(idx_hbm.at[pl.ds(base, b_per_w)], idx_v)
        copies = []
        off = 0
        for _ in range(num_chunks):
            n = min(CHUNK, b_per_w - off)
            c = pltpu.make_async_copy(
                table_hbm.at[idx_v.at[pl.ds(off, n)]],
                rows_v.at[pl.ds(off, n)],
                sem,
            )
            c.start()
            copies.append(c)
            off += n
        for c in copies:
            c.wait()
        pltpu.sync_copy(rows_v, out_hbm.at[pl.ds(base, b_per_w)])

    return k(table, idx)


def _expand_body(x_ref, o_ref):
    bt = x_ref.shape[0]
    bits = lax.bitcast_convert_type(x_ref[...], jnp.int32)
    parts = [
        jnp.broadcast_to(bits[:, e:e + 1], (bt, BITS)) for e in range(EMBED)
    ]
    big = jnp.concatenate(parts, axis=1)
    lane = lax.broadcasted_iota(jnp.int32, (bt, EMBED * BITS), 1)
    mask = jnp.left_shift(jnp.int32(1), 31 - (lane & (BITS - 1)))
    dense = ((big & mask) != 0).astype(jnp.float32)
    for e in range(EMBED):
        o_ref[pl.Slice(e, bt, EMBED), :] = dense[:, e * BITS:(e + 1) * BITS]


def _tc_expand(gathered, n_tokens):
    bt = 64
    grid = (n_tokens // bt,)
    return pl.pallas_call(
        _expand_body,
        grid=grid,
        in_specs=[pl.BlockSpec((bt, EMBED), lambda i: (i, 0))],
        out_specs=pl.BlockSpec((bt * EMBED, BITS), lambda i: (i, 0)),
        out_shape=jax.ShapeDtypeStruct(
            (n_tokens * EMBED, BITS), jnp.float32),
    )(gathered)


def kernel(token_ids, weight_float):
    batch_shape = token_ids.shape
    flat_ids = token_ids.reshape(-1).astype(jnp.int32)
    n_tokens = flat_ids.shape[0]
    gathered = _sc_gather(weight_float, flat_ids, n_tokens)
    out = _tc_expand(gathered, n_tokens)
    return out.reshape(batch_shape + (EMBED, BITS))


# speedup vs baseline: 5.5578x; 5.5578x over previous
import functools

import jax
import jax.numpy as jnp
from jax import lax
from jax.experimental import pallas as pl
from jax.experimental.pallas import tpu as pltpu
from jax.experimental.pallas import tpu_sc as plsc

EMBED = 16
BITS = 32
NUM_WORKERS = 32
CHUNK = 128


def _sc_gather(table, idx, n_tokens):
    b_per_w = n_tokens // NUM_WORKERS
    num_chunks = (b_per_w + CHUNK - 1) // CHUNK
    mesh = plsc.VectorSubcoreMesh(core_axis_name="c", subcore_axis_name="s")

    @functools.partial(
        pl.kernel,
        mesh=mesh,
        out_type=jax.ShapeDtypeStruct((n_tokens, EMBED), jnp.float32),
        scratch_types=[
            pltpu.VMEM((b_per_w,), jnp.int32),
            pltpu.VMEM((b_per_w, EMBED), jnp.float32),
            pltpu.SemaphoreType.DMA,
        ],
        compiler_params=pltpu.CompilerParams(use_tc_tiling_on_sc=False),
    )
    def k(table_hbm, idx_hbm, out_hbm, idx_v, rows_v, sem):
        wid = lax.axis_index("s") * 2 + lax.axis_index("c")
        base = wid * b_per_w
        pltpu.sync_copy(idx_hbm.at[pl.ds(base, b_per_w)], idx_v)
        copies = []
        off = 0
        for _ in range(num_chunks):
            n = min(CHUNK, b_per_w - off)
            c = pltpu.make_async_copy(
                table_hbm.at[idx_v.at[pl.ds(off, n)]],
                rows_v.at[pl.ds(off, n)],
                sem,
            )
            c.start()
            copies.append(c)
            off += n
        for c in copies:
            c.wait()
        pltpu.sync_copy(rows_v, out_hbm.at[pl.ds(base, b_per_w)])

    return k(table, idx)


def _expand_body(x_ref, o_ref):
    nb = x_ref.shape[1]
    xt = lax.transpose(x_ref[0], (1, 0))
    bits = lax.bitcast_convert_type(xt, jnp.int32)
    b_iota = lax.broadcasted_iota(jnp.int32, (BITS, nb), 0)
    mask = jnp.left_shift(jnp.int32(1), 31 - b_iota)
    for e in range(EMBED):
        rep = jnp.broadcast_to(bits[e:e + 1, :], (BITS, nb))
        o_ref[0, e] = ((rep & mask) != 0).astype(jnp.float32)


def _tc_expand(gathered_t, n_seq, n_batch):
    return pl.pallas_call(
        _expand_body,
        grid=(n_seq,),
        in_specs=[pl.BlockSpec((1, n_batch, EMBED), lambda s: (s, 0, 0))],
        out_specs=pl.BlockSpec(
            (1, EMBED, BITS, n_batch), lambda s: (s, 0, 0, 0)),
        out_shape=jax.ShapeDtypeStruct(
            (n_seq, EMBED, BITS, n_batch), jnp.float32),
    )(gathered_t)


def kernel(token_ids, weight_float):
    n_batch, n_seq = token_ids.shape
    n_tokens = n_batch * n_seq
    flat_ids = token_ids.T.reshape(-1).astype(jnp.int32)
    gathered = _sc_gather(weight_float, flat_ids, n_tokens)
    gathered_t = gathered.reshape(n_seq, n_batch, EMBED)
    out_t = _tc_expand(gathered_t, n_seq, n_batch)
    return jnp.transpose(out_t, (3, 0, 1, 2))


# speedup vs baseline: 5.6346x; 1.0138x over previous
import functools

import jax
import jax.numpy as jnp
from jax import lax
from jax.experimental import pallas as pl
from jax.experimental.pallas import tpu as pltpu
from jax.experimental.pallas import tpu_sc as plsc

EMBED = 16
BITS = 32
NUM_WORKERS = 32
CHUNK = 128


def _sc_gather_planes(table_flat, ids_t, vocab, n_seq, n_batch):
    n_planes = n_seq * EMBED
    planes_per_w = n_planes // NUM_WORKERS
    chunks_per_plane = n_batch // CHUNK
    mesh = plsc.VectorSubcoreMesh(core_axis_name="c", subcore_axis_name="s")

    @functools.partial(
        pl.kernel,
        mesh=mesh,
        out_type=jax.ShapeDtypeStruct((n_planes * n_batch,), jnp.float32),
        scratch_types=[
            pltpu.VMEM((n_batch,), jnp.int32),
            pltpu.VMEM((n_batch,), jnp.float32),
            pltpu.SemaphoreType.DMA,
        ],
        compiler_params=pltpu.CompilerParams(use_tc_tiling_on_sc=False),
    )
    def k(table_hbm, ids_hbm, out_hbm, idx_v, plane_v, sem):
        wid = lax.axis_index("s") * 2 + lax.axis_index("c")
        p0 = wid * planes_per_w

        def one_plane(i):
            p = p0 + i
            s = p // EMBED
            e = p % EMBED
            pltpu.sync_copy(ids_hbm.at[pl.ds(s * n_batch, n_batch)], idx_v)
            table_e = table_hbm.at[pl.ds(e * vocab, vocab)]
            copies = []
            for c in range(chunks_per_plane):
                cp = pltpu.make_async_copy(
                    table_e.at[idx_v.at[pl.ds(c * CHUNK, CHUNK)]],
                    plane_v.at[pl.ds(c * CHUNK, CHUNK)],
                    sem,
                )
                cp.start()
                copies.append(cp)
            for cp in copies:
                cp.wait()
            pltpu.sync_copy(plane_v, out_hbm.at[pl.ds(p * n_batch, n_batch)])

        pl.loop(0, planes_per_w)(one_plane)

    return k(table_flat, ids_t)


def _expand_body(x_ref, o_ref):
    n_sub = x_ref.shape[0] // EMBED
    nb = n_sub * 128
    bits = lax.bitcast_convert_type(x_ref[...], jnp.int32)
    b_iota = lax.broadcasted_iota(jnp.int32, (BITS, 128), 0)
    mask = jnp.left_shift(jnp.int32(1), 31 - b_iota)
    for e in range(EMBED):
        pieces = [
            ((jnp.broadcast_to(bits[n_sub * e + c:n_sub * e + c + 1, :],
                               (BITS, 128)) & mask) != 0).astype(jnp.float32)
            for c in range(n_sub)
        ]
        o_ref[0, e] = jnp.concatenate(pieces, axis=1)


def _tc_expand(planes_2d, n_seq, n_batch):
    rows_per_s = EMBED * n_batch // 128
    return pl.pallas_call(
        _expand_body,
        grid=(n_seq,),
        in_specs=[pl.BlockSpec((rows_per_s, 128), lambda s: (s, 0))],
        out_specs=pl.BlockSpec(
            (1, EMBED, BITS, n_batch), lambda s: (s, 0, 0, 0)),
        out_shape=jax.ShapeDtypeStruct(
            (n_seq, EMBED, BITS, n_batch), jnp.float32),
    )(planes_2d)


def kernel(token_ids, weight_float):
    n_batch, n_seq = token_ids.shape
    vocab = weight_float.shape[0]
    ids_t = token_ids.T.reshape(-1).astype(jnp.int32)
    table_flat = weight_float.T.reshape(-1)
    planes = _sc_gather_planes(table_flat, ids_t, vocab, n_seq, n_batch)
    planes_2d = planes.reshape(n_seq * EMBED * n_batch // 128, 128)
    out_t = _tc_expand(planes_2d, n_seq, n_batch)
    return jnp.transpose(out_t, (3, 0, 1, 2))


# speedup vs baseline: 6.7191x; 1.1925x over previous
import functools

import jax
import jax.numpy as jnp
from jax import lax
from jax.experimental import pallas as pl
from jax.experimental.pallas import tpu as pltpu
from jax.experimental.pallas import tpu_sc as plsc

EMBED = 16
BITS = 32
NUM_WORKERS = 32
CHUNK = 128


def _sc_gather_planes(table_flat, ids_t, vocab, n_seq, n_batch):
    chunks_per_plane = n_batch // CHUNK
    s_rounds = (n_seq + NUM_WORKERS - 1) // NUM_WORKERS
    blk = EMBED * n_batch
    mesh = plsc.VectorSubcoreMesh(core_axis_name="c", subcore_axis_name="s")

    @functools.partial(
        pl.kernel,
        mesh=mesh,
        out_type=jax.ShapeDtypeStruct((n_seq * blk,), jnp.float32),
        scratch_types=[
            pltpu.VMEM((n_batch,), jnp.int32),
            pltpu.VMEM((blk,), jnp.float32),
            pltpu.SemaphoreType.DMA,
        ],
        compiler_params=pltpu.CompilerParams(use_tc_tiling_on_sc=False),
    )
    def k(table_hbm, ids_hbm, out_hbm, idx_v, planes_v, sem):
        wid = lax.axis_index("s") * 2 + lax.axis_index("c")

        for rep in range(s_rounds):
            s = wid + NUM_WORKERS * rep

            @pl.when(s < n_seq)
            def _():
                pltpu.sync_copy(ids_hbm.at[pl.ds(s * n_batch, n_batch)],
                                idx_v)

                def one_plane(e):
                    table_e = table_hbm.at[pl.ds(e * vocab, vocab)]
                    for c in range(chunks_per_plane):
                        pltpu.make_async_copy(
                            table_e.at[idx_v.at[pl.ds(c * CHUNK, CHUNK)]],
                            planes_v.at[pl.ds(e * n_batch + c * CHUNK,
                                              CHUNK)],
                            sem,
                        ).start()

                pl.loop(0, EMBED)(one_plane)
                pltpu.make_async_copy(
                    out_hbm.at[pl.ds(0, blk)], planes_v, sem).wait()
                pltpu.sync_copy(planes_v,
                                out_hbm.at[pl.ds(s * blk, blk)])

    return k(table_flat, ids_t)


def _expand_body(x_ref, o_ref):
    n_sub = x_ref.shape[0] // EMBED
    nb = n_sub * 128
    bits = lax.bitcast_convert_type(x_ref[...], jnp.int32)
    b_iota = lax.broadcasted_iota(jnp.int32, (BITS, 128), 0)
    mask = jnp.left_shift(jnp.int32(1), 31 - b_iota)
    for e in range(EMBED):
        pieces = [
            ((jnp.broadcast_to(bits[n_sub * e + c:n_sub * e + c + 1, :],
                               (BITS, 128)) & mask) != 0).astype(jnp.float32)
            for c in range(n_sub)
        ]
        o_ref[0, e] = jnp.concatenate(pieces, axis=1)


def _tc_expand(planes_2d, n_seq, n_batch):
    rows_per_s = EMBED * n_batch // 128
    return pl.pallas_call(
        _expand_body,
        grid=(n_seq,),
        in_specs=[pl.BlockSpec((rows_per_s, 128), lambda s: (s, 0))],
        out_specs=pl.BlockSpec(
            (1, EMBED, BITS, n_batch), lambda s: (s, 0, 0, 0)),
        out_shape=jax.ShapeDtypeStruct(
            (n_seq, EMBED, BITS, n_batch), jnp.float32),
    )(planes_2d)


def kernel(token_ids, weight_float):
    n_batch, n_seq = token_ids.shape
    vocab = weight_float.shape[0]
    ids_t = token_ids.T.reshape(-1).astype(jnp.int32)
    table_flat = weight_float.T.reshape(-1)
    planes = _sc_gather_planes(table_flat, ids_t, vocab, n_seq, n_batch)
    planes_2d = planes.reshape(n_seq * EMBED * n_batch // 128, 128)
    out_t = _tc_expand(planes_2d, n_seq, n_batch)
    return jnp.transpose(out_t, (3, 0, 1, 2))


# speedup vs baseline: 7.1170x; 1.0592x over previous
import functools

import jax
import jax.numpy as jnp
from jax import lax
from jax.experimental import pallas as pl
from jax.experimental.pallas import tpu as pltpu
from jax.experimental.pallas import tpu_sc as plsc

EMBED = 16
BITS = 32
NUM_WORKERS = 32
CHUNK = 128


def _sc_gather_planes(table_flat, ids_t, vocab, n_seq, n_batch):
    n_halves = NUM_WORKERS // EMBED
    s_per_h = n_seq // n_halves
    lanes = 16
    mesh = plsc.VectorSubcoreMesh(core_axis_name="c", subcore_axis_name="s")

    @functools.partial(
        pl.kernel,
        mesh=mesh,
        out_type=jax.ShapeDtypeStruct((n_seq * EMBED * n_batch,),
                                      jnp.float32),
        scratch_types=[
            pltpu.VMEM((vocab,), jnp.float32),
            pltpu.VMEM((n_batch,), jnp.int32),
            pltpu.VMEM((n_batch,), jnp.float32),
            pltpu.SemaphoreType.DMA,
        ],
        compiler_params=pltpu.CompilerParams(
            use_tc_tiling_on_sc=False, needs_layout_passes=False),
    )
    def k(table_hbm, ids_hbm, out_hbm, plane_v, idx_v, vals_v, sem):
        wid = lax.axis_index("s") * 2 + lax.axis_index("c")
        e = wid % EMBED
        h = wid // EMBED
        pltpu.sync_copy(table_hbm.at[pl.ds(e * vocab, vocab)], plane_v)

        def one_s(i):
            s = h * s_per_h + i
            pltpu.sync_copy(ids_hbm.at[pl.ds(s * n_batch, n_batch)], idx_v)

            def one_vec(j):
                idx16 = idx_v[pl.ds(j * lanes, lanes)]
                vals_v[pl.ds(j * lanes, lanes)] = plsc.load_gather(
                    plane_v, [idx16])

            pl.loop(0, n_batch // lanes)(one_vec)
            pltpu.sync_copy(
                vals_v,
                out_hbm.at[pl.ds((s * EMBED + e) * n_batch, n_batch)])

        pl.loop(0, s_per_h)(one_s)

    return k(table_flat, ids_t)


def _expand_body(x_ref, o_ref):
    n_sub = x_ref.shape[0] // EMBED
    nb = n_sub * 128
    bits = lax.bitcast_convert_type(x_ref[...], jnp.int32)
    b_iota = lax.broadcasted_iota(jnp.int32, (BITS, 128), 0)
    mask = jnp.left_shift(jnp.int32(1), 31 - b_iota)
    for e in range(EMBED):
        pieces = [
            ((jnp.broadcast_to(bits[n_sub * e + c:n_sub * e + c + 1, :],
                               (BITS, 128)) & mask) != 0).astype(jnp.float32)
            for c in range(n_sub)
        ]
        o_ref[0, e] = jnp.concatenate(pieces, axis=1)


def _tc_expand(planes_2d, n_seq, n_batch):
    rows_per_s = EMBED * n_batch // 128
    return pl.pallas_call(
        _expand_body,
        grid=(n_seq,),
        in_specs=[pl.BlockSpec((rows_per_s, 128), lambda s: (s, 0))],
        out_specs=pl.BlockSpec(
            (1, EMBED, BITS, n_batch), lambda s: (s, 0, 0, 0)),
        out_shape=jax.ShapeDtypeStruct(
            (n_seq, EMBED, BITS, n_batch), jnp.float32),
    )(planes_2d)


def kernel(token_ids, weight_float):
    n_batch, n_seq = token_ids.shape
    vocab = weight_float.shape[0]
    ids_t = token_ids.T.reshape(-1).astype(jnp.int32)
    table_flat = weight_float.T.reshape(-1)
    planes = _sc_gather_planes(table_flat, ids_t, vocab, n_seq, n_batch)
    planes_2d = planes.reshape(n_seq * EMBED * n_batch // 128, 128)
    out_t = _tc_expand(planes_2d, n_seq, n_batch)
    return jnp.transpose(out_t, (3, 0, 1, 2))


# speedup vs baseline: 8.0628x; 1.1329x over previous
import functools

import jax
import jax.numpy as jnp
from jax import lax
from jax.experimental import pallas as pl
from jax.experimental.pallas import tpu as pltpu
from jax.experimental.pallas import tpu_sc as plsc

EMBED = 16
BITS = 32
NUM_WORKERS = 32
CHUNK = 128


def _sc_gather_planes(table_flat, ids_t, vocab, n_seq, n_batch):
    n_halves = NUM_WORKERS // EMBED
    s_per_h = n_seq // n_halves
    lanes = 16
    mesh = plsc.VectorSubcoreMesh(core_axis_name="c", subcore_axis_name="s")

    @functools.partial(
        pl.kernel,
        mesh=mesh,
        out_type=jax.ShapeDtypeStruct((n_seq * EMBED * n_batch,),
                                      jnp.float32),
        scratch_types=[
            pltpu.VMEM((vocab,), jnp.float32),
            pltpu.VMEM((s_per_h * n_batch,), jnp.int32),
            pltpu.VMEM((2, n_batch), jnp.float32),
            pltpu.SemaphoreType.DMA,
            pltpu.SemaphoreType.DMA,
        ],
        compiler_params=pltpu.CompilerParams(
            use_tc_tiling_on_sc=False, needs_layout_passes=False),
    )
    def k(table_hbm, ids_hbm, out_hbm, plane_v, ids_v, vals_v, sem, wsem):
        wid = lax.axis_index("s") * 2 + lax.axis_index("c")
        e = wid % EMBED
        h = wid // EMBED
        c1 = pltpu.make_async_copy(
            table_hbm.at[pl.ds(e * vocab, vocab)], plane_v, sem)
        c1.start()
        c2 = pltpu.make_async_copy(
            ids_hbm.at[pl.ds(h * s_per_h * n_batch, s_per_h * n_batch)],
            ids_v, sem)
        c2.start()
        c1.wait()
        c2.wait()

        def one_s(i, _):
            s = h * s_per_h + i
            buf = lax.rem(i, 2)

            @pl.when(i >= 2)
            def _():
                pltpu.make_async_copy(
                    out_hbm.at[pl.ds(0, n_batch)],
                    vals_v.at[buf], wsem).wait()

            def one_vec(j, _):
                idx16 = ids_v[pl.ds(i * n_batch + j * lanes, lanes)]
                vals_v[buf, pl.ds(j * lanes, lanes)] = plsc.load_gather(
                    plane_v, [idx16])
                return ()

            lax.fori_loop(0, n_batch // lanes, one_vec, (), unroll=4)
            pltpu.make_async_copy(
                vals_v.at[buf],
                out_hbm.at[pl.ds((s * EMBED + e) * n_batch, n_batch)],
                wsem,
            ).start()
            return ()

        lax.fori_loop(0, s_per_h, one_s, ())
        for b in range(2):
            pltpu.make_async_copy(
                out_hbm.at[pl.ds(0, n_batch)], vals_v.at[b], wsem).wait()

    return k(table_flat, ids_t)


def _expand_body(x_ref, o_ref):
    n_sub = x_ref.shape[0] // EMBED
    nb = n_sub * 128
    bits = lax.bitcast_convert_type(x_ref[...], jnp.int32)
    b_iota = lax.broadcasted_iota(jnp.int32, (BITS, 128), 0)
    mask = jnp.left_shift(jnp.int32(1), 31 - b_iota)
    for e in range(EMBED):
        pieces = [
            ((jnp.broadcast_to(bits[n_sub * e + c:n_sub * e + c + 1, :],
                               (BITS, 128)) & mask) != 0).astype(jnp.float32)
            for c in range(n_sub)
        ]
        o_ref[0, e] = jnp.concatenate(pieces, axis=1)


def _tc_expand(planes_2d, n_seq, n_batch):
    rows_per_s = EMBED * n_batch // 128
    return pl.pallas_call(
        _expand_body,
        grid=(n_seq,),
        in_specs=[pl.BlockSpec((rows_per_s, 128), lambda s: (s, 0))],
        out_specs=pl.BlockSpec(
            (1, EMBED, BITS, n_batch), lambda s: (s, 0, 0, 0)),
        out_shape=jax.ShapeDtypeStruct(
            (n_seq, EMBED, BITS, n_batch), jnp.float32),
    )(planes_2d)


def kernel(token_ids, weight_float):
    n_batch, n_seq = token_ids.shape
    vocab = weight_float.shape[0]
    ids_t = token_ids.T.reshape(-1).astype(jnp.int32)
    table_flat = weight_float.T.reshape(-1)
    planes = _sc_gather_planes(table_flat, ids_t, vocab, n_seq, n_batch)
    planes_2d = planes.reshape(n_seq * EMBED * n_batch // 128, 128)
    out_t = _tc_expand(planes_2d, n_seq, n_batch)
    return jnp.transpose(out_t, (3, 0, 1, 2))


# speedup vs baseline: 8.7904x; 1.0902x over previous
import functools

import jax
import jax.numpy as jnp
from jax import lax
from jax.experimental import pallas as pl
from jax.experimental.pallas import tpu as pltpu
from jax.experimental.pallas import tpu_sc as plsc

EMBED = 16
BITS = 32
NUM_WORKERS = 32
CHUNK = 128


def _sc_gather_planes(table_t, ids_t, vocab, n_seq, n_batch):
    n_halves = NUM_WORKERS // EMBED
    s_per_h = n_seq // n_halves
    lanes = 16
    mesh = plsc.VectorSubcoreMesh(core_axis_name="c", subcore_axis_name="s")

    @functools.partial(
        pl.kernel,
        mesh=mesh,
        out_type=jax.ShapeDtypeStruct((n_seq * EMBED * n_batch,),
                                      jnp.float32),
        scratch_types=[
            pltpu.VMEM((vocab,), jnp.float32),
            pltpu.VMEM((s_per_h * n_batch,), jnp.int32),
            pltpu.VMEM((2, n_batch), jnp.float32),
            pltpu.SemaphoreType.DMA,
            pltpu.SemaphoreType.DMA,
        ],
        compiler_params=pltpu.CompilerParams(
            use_tc_tiling_on_sc=True, needs_layout_passes=False),
    )
    def k(table_hbm, ids_hbm, out_hbm, plane_v, ids_v, vals_v, sem, wsem):
        wid = lax.axis_index("s") * 2 + lax.axis_index("c")
        e = wid % EMBED
        h = wid // EMBED
        c1 = pltpu.make_async_copy(table_hbm.at[e], plane_v, sem)
        c1.start()
        c2 = pltpu.make_async_copy(
            ids_hbm.at[pl.ds(h * s_per_h * n_batch, s_per_h * n_batch)],
            ids_v, sem)
        c2.start()
        c1.wait()
        c2.wait()

        def one_s(i, _):
            s = h * s_per_h + i
            buf = lax.rem(i, 2)

            @pl.when(i >= 2)
            def _():
                pltpu.make_async_copy(
                    out_hbm.at[pl.ds(0, n_batch)],
                    vals_v.at[buf], wsem).wait()

            def one_vec(j, _):
                idx16 = ids_v[pl.ds(i * n_batch + j * lanes, lanes)]
                vals_v[buf, pl.ds(j * lanes, lanes)] = plsc.load_gather(
                    plane_v, [idx16])
                return ()

            lax.fori_loop(0, n_batch // lanes, one_vec, (), unroll=8)
            pltpu.make_async_copy(
                vals_v.at[buf],
                out_hbm.at[pl.ds((s * EMBED + e) * n_batch, n_batch)],
                wsem,
            ).start()
            return ()

        lax.fori_loop(0, s_per_h, one_s, ())
        for b in range(2):
            pltpu.make_async_copy(
                out_hbm.at[pl.ds(0, n_batch)], vals_v.at[b], wsem).wait()

    return k(table_t, ids_t)


def _expand_body(x_ref, o_ref):
    n_sub = x_ref.shape[0] // EMBED
    nb = n_sub * 128
    bits = lax.bitcast_convert_type(x_ref[...], jnp.int32)
    b_iota = lax.broadcasted_iota(jnp.int32, (BITS, 128), 0)
    mask = jnp.left_shift(jnp.int32(1), 31 - b_iota)
    for e in range(EMBED):
        pieces = [
            ((jnp.broadcast_to(bits[n_sub * e + c:n_sub * e + c + 1, :],
                               (BITS, 128)) & mask) != 0).astype(jnp.float32)
            for c in range(n_sub)
        ]
        o_ref[0, e] = jnp.concatenate(pieces, axis=1)


def _tc_expand(planes_2d, n_seq, n_batch):
    rows_per_s = EMBED * n_batch // 128
    return pl.pallas_call(
        _expand_body,
        grid=(n_seq,),
        in_specs=[pl.BlockSpec((rows_per_s, 128), lambda s: (s, 0))],
        out_specs=pl.BlockSpec(
            (1, EMBED, BITS, n_batch), lambda s: (s, 0, 0, 0)),
        out_shape=jax.ShapeDtypeStruct(
            (n_seq, EMBED, BITS, n_batch), jnp.float32),
    )(planes_2d)


def kernel(token_ids, weight_float):
    n_batch, n_seq = token_ids.shape
    vocab = weight_float.shape[0]
    ids_t = token_ids.T.reshape(-1).astype(jnp.int32)
    table_t = weight_float.T
    planes = _sc_gather_planes(table_t, ids_t, vocab, n_seq, n_batch)
    planes_2d = planes.reshape(n_seq * EMBED * n_batch // 128, 128)
    out_t = _tc_expand(planes_2d, n_seq, n_batch)
    return jnp.transpose(out_t, (3, 0, 1, 2))
